# baseline (speedup 1.0000x reference)
import functools

import jax
import jax.numpy as jnp
from jax import lax
from jax.experimental import pallas as pl
from jax.experimental.pallas import tpu as pltpu, tpu_sc as plsc

_NC = 2
_NS = 16
_NW = _NC * _NS
_L = 16
_WINDOW_GROUPS = 4


@jax.jit
def _embed_lookup(x, table):
    B, = x.shape
    V, D = table.shape
    b_per_w = B // _NW
    n_groups = b_per_w // _L

    mesh = plsc.VectorSubcoreMesh(core_axis_name="c", subcore_axis_name="s")

    @functools.partial(
        pl.kernel,
        mesh=mesh,
        out_type=jax.ShapeDtypeStruct((B, D), jnp.float32),
        scratch_types=[
            pltpu.VMEM((b_per_w,), jnp.int32),
            pltpu.SemaphoreType.DMA,
        ],
    )
    def k(x_hbm, table_hbm, out_hbm, idx_v, sem):
        wid = lax.axis_index("s") * _NC + lax.axis_index("c")
        base = wid * b_per_w

        pltpu.sync_copy(x_hbm.at[pl.ds(base, b_per_w)], idx_v)

        def issue_group(g):
            v = idx_v[pl.ds(g * _L, _L)]
            for j in range(_L):
                pltpu.async_copy(
                    table_hbm.at[v[j]], out_hbm.at[base + g * _L + j], sem
                )

        def wait_group():
            for _ in range(_L):
                pltpu.make_async_copy(
                    table_hbm.at[0], out_hbm.at[base], sem
                ).wait()

        def body(g, _):
            issue_group(g)

            @pl.when(g >= _WINDOW_GROUPS)
            def _drain_one():
                wait_group()

            return 0

        lax.fori_loop(0, n_groups, body, 0)

        def drain(i, _):
            wait_group()
            return 0

        lax.fori_loop(0, _WINDOW_GROUPS, drain, 0)

    return k(x, table)


def kernel(x, table):
    return _embed_lookup(x.astype(jnp.int32), table)


# speedup vs baseline: 5.3007x; 5.3007x over previous
import functools

import jax
import jax.numpy as jnp
from jax import lax
from jax.experimental import pallas as pl
from jax.experimental.pallas import tpu as pltpu, tpu_sc as plsc

_NC = 2
_NS = 16
_NW = _NC * _NS
_L = 16
_CHUNK = 2048


@jax.jit
def _embed_lookup(x, table):
    B, = x.shape
    V, D = table.shape
    rows_per_w = D // _NW
    n_chunks = B // _CHUNK

    tableT = table.T
    mesh = plsc.VectorSubcoreMesh(core_axis_name="c", subcore_axis_name="s")

    @functools.partial(
        pl.kernel,
        mesh=mesh,
        compiler_params=pltpu.CompilerParams(needs_layout_passes=False),
        out_type=jax.ShapeDtypeStruct((D, B), jnp.float32),
        scratch_types=[
            pltpu.VMEM((V,), jnp.float32),
            pltpu.VMEM((_CHUNK,), jnp.int32),
            pltpu.VMEM((_CHUNK,), jnp.float32),
        ],
    )
    def k(x_hbm, tableT_hbm, outT_hbm, row_v, idx_v, val_v):
        wid = lax.axis_index("s") * _NC + lax.axis_index("c")

        for p in range(rows_per_w):
            i = wid + p * _NW
            pltpu.sync_copy(tableT_hbm.at[i], row_v)
            for c in range(n_chunks):
                pltpu.sync_copy(x_hbm.at[pl.ds(c * _CHUNK, _CHUNK)], idx_v)

                def gbody(g, _):
                    iv = idx_v[pl.ds(g * _L, _L)]
                    val_v[pl.ds(g * _L, _L)] = plsc.load_gather(row_v, [iv])
                    return 0

                lax.fori_loop(0, _CHUNK // _L, gbody, 0)
                pltpu.sync_copy(val_v, outT_hbm.at[i, pl.ds(c * _CHUNK, _CHUNK)])

    outT = k(x, tableT)
    return outT.T


def kernel(x, table):
    return _embed_lookup(x.astype(jnp.int32), table)
